# baseline (speedup 1.0000x reference)
import functools

import jax
import jax.numpy as jnp
from jax import lax
from jax.experimental import pallas as pl
from jax.experimental.pallas import tpu as pltpu
from jax.experimental.pallas import tpu_sc as plsc

_SIZE = 100000
_DIM = 32
_BATCH = 1024
_TILE = 10000
_NTILES = _SIZE // _TILE
_LANES = 128
_RB = 128
_NRB = _BATCH // _RB

_NC = 2
_NS = 16
_NW = _NC * _NS
_BPW = _BATCH // _NW


def _argmax_body(x_ref, cand_ref, qidx_ref, bestsim_ref, simmean_ref,
                 bv_ref, bi_ref):
    i = pl.program_id(0)
    r = pl.program_id(1)

    xx = x_ref[...]
    xnorm = jnp.sqrt(jnp.sum(xx * xx, axis=1, keepdims=True))
    nx = xx / jnp.maximum(xnorm, 1e-12)

    ones = jnp.ones((8, _DIM), jnp.float32)
    m = jnp.full((_RB, _LANES), -jnp.inf, jnp.float32)
    tc = jnp.zeros((_RB, _LANES), jnp.int32)
    nchunks = -(-_TILE // _LANES)
    for j in range(nchunks):
        w = min(_LANES, _TILE - j * _LANES)
        c = cand_ref[pl.ds(j * _LANES, w), :_DIM]
        csq = c * c
        n2 = lax.dot_general(ones, csq, (((1,), (1,)), ((), ())),
                             precision=lax.Precision.HIGHEST,
                             preferred_element_type=jnp.float32)[0:1]
        inv = 1.0 / jnp.maximum(jnp.sqrt(n2), 1e-12)
        s = lax.dot_general(nx, c, (((1,), (1,)), ((), ())),
                            precision=lax.Precision.HIGHEST,
                            preferred_element_type=jnp.float32)
        s = s * inv
        if w < _LANES:
            s = jnp.concatenate(
                [s, jnp.full((_RB, _LANES - w), -jnp.inf, jnp.float32)],
                axis=1)
        upd = s > m
        m = jnp.where(upd, s, m)
        tc = jnp.where(upd, j, tc)

    tmax = jnp.max(m, axis=1, keepdims=True)
    col = tc * _LANES + lax.broadcasted_iota(jnp.int32, (_RB, _LANES), 1)
    cand = jnp.where(m == tmax, col, jnp.int32(2**31 - 1))
    targ = jnp.min(cand, axis=1, keepdims=True)

    rows = pl.ds(r * _RB, _RB)

    @pl.when(i == 0)
    def _init():
        bv_ref[rows, :] = jnp.full((_RB, 1), -jnp.inf, jnp.float32)
        bi_ref[rows, :] = jnp.zeros((_RB, 1), jnp.int32)

    upd2 = tmax > bv_ref[rows, :]
    bi_ref[rows, :] = jnp.where(upd2, i * _TILE + targ, bi_ref[rows, :])
    bv_ref[rows, :] = jnp.where(upd2, tmax, bv_ref[rows, :])

    @pl.when(i == _NTILES - 1)
    def _fin():
        qidx_ref[rows, :] = bi_ref[rows, :]
        bestsim_ref[rows, :] = bv_ref[rows, :]

    @pl.when((i == _NTILES - 1) & (r == _NRB - 1))
    def _mean():
        simmean_ref[...] = jnp.full(
            (8, _LANES), jnp.sum(bv_ref[...]) / _BATCH, jnp.float32)


def _tc_argmax(x, qx2):
    return pl.pallas_call(
        _argmax_body,
        grid=(_NTILES, _NRB),
        in_specs=[
            pl.BlockSpec((_RB, _DIM), lambda i, r: (r, 0)),
            pl.BlockSpec((_TILE, 2 * _DIM), lambda i, r: (i, 0)),
        ],
        out_specs=[
            pl.BlockSpec((_BATCH, 1), lambda i, r: (0, 0)),
            pl.BlockSpec((_BATCH, 1), lambda i, r: (0, 0)),
            pl.BlockSpec((8, _LANES), lambda i, r: (0, 0)),
        ],
        out_shape=[
            jax.ShapeDtypeStruct((_BATCH, 1), jnp.int32),
            jax.ShapeDtypeStruct((_BATCH, 1), jnp.float32),
            jax.ShapeDtypeStruct((8, _LANES), jnp.float32),
        ],
        scratch_shapes=[
            pltpu.VMEM((_BATCH, 1), jnp.float32),
            pltpu.VMEM((_BATCH, 1), jnp.int32),
        ],
    )(x, qx2)


def _sc_gather_body(qx_hbm, age_hbm, idx_hbm, rows_out, age_out, psum_out,
                    idx_v, rows_v, age_v, psum_v, sem):
    wid = lax.axis_index("s") * _NC + lax.axis_index("c")
    base = wid * _BPW
    pltpu.sync_copy(idx_hbm.at[pl.ds(base, _BPW)], idx_v)
    pltpu.async_copy(qx_hbm.at[idx_v], rows_v, sem).wait()
    pltpu.async_copy(age_hbm.at[idx_v], age_v, sem).wait()
    pltpu.sync_copy(rows_v, rows_out.at[pl.ds(base, _BPW)])
    pltpu.sync_copy(age_v, age_out.at[pl.ds(base, _BPW)])
    a = jnp.zeros((16,), jnp.float32)
    for k in range(_BPW // 16):
        a = a + age_v[pl.ds(k * 16, 16)].astype(jnp.float32)
    psum_v[0, :] = a
    pltpu.sync_copy(psum_v, psum_out.at[pl.ds(wid, 1)])


def _sc_gather(queue_x, queue_age, nn_qidx):
    mesh = plsc.VectorSubcoreMesh(core_axis_name="c", subcore_axis_name="s")
    fn = functools.partial(
        pl.kernel,
        mesh=mesh,
        out_type=[
            jax.ShapeDtypeStruct((_BATCH, 2 * _DIM), jnp.float32),
            jax.ShapeDtypeStruct((_BATCH,), jnp.int32),
            jax.ShapeDtypeStruct((_NW, 16), jnp.float32),
        ],
        scratch_types=[
            pltpu.VMEM((_BPW,), jnp.int32),
            pltpu.VMEM((_BPW, 2 * _DIM), jnp.float32),
            pltpu.VMEM((_BPW,), jnp.int32),
            pltpu.VMEM((1, 16), jnp.float32),
            pltpu.SemaphoreType.DMA,
        ],
        compiler_params=pltpu.CompilerParams(use_tc_tiling_on_sc=False),
    )(_sc_gather_body)
    return fn(queue_x, queue_age, nn_qidx)


def kernel(x, idx, queue_x, queue_age):
    qx2 = queue_x.reshape(_SIZE, 2 * _DIM)
    nn_qidx2, best_sim, sim_mean = _tc_argmax(x, qx2)
    nn_qidx = nn_qidx2.reshape(_BATCH)
    rows, nn_age, age_psums = _sc_gather(qx2, queue_age, nn_qidx)
    nn_x = rows[:, :_DIM]
    age_mean = jnp.sum(age_psums) / _BATCH
    return (nn_x, sim_mean[0, 0], age_mean)


# speedup vs baseline: 8.2450x; 8.2450x over previous
import functools

import jax
import jax.numpy as jnp
from jax import lax
from jax.experimental import pallas as pl
from jax.experimental.pallas import tpu as pltpu
from jax.experimental.pallas import tpu_sc as plsc

_SIZE = 100000
_DIM = 32
_BATCH = 1024
_TILE = 10000
_NTILES = _SIZE // _TILE
_LANES = 128
_RB = 1024
_NRB = _BATCH // _RB

_NC = 2
_NS = 16
_NW = _NC * _NS
_BPW = _BATCH // _NW


def _argmax_body(x_ref, cand_ref, qidx_ref, bestsim_ref, simmean_ref,
                 bv_ref, bi_ref, ncn_ref):
    i = pl.program_id(0)
    r = pl.program_id(1)

    xx = x_ref[...]
    xnorm = jnp.sqrt(jnp.sum(xx * xx, axis=1, keepdims=True))
    nx = xx / jnp.maximum(xnorm, 1e-12)

    @pl.when(r == 0)
    def _prep():
        c = cand_ref[:, :_DIM]
        cnorm = jnp.sqrt(jnp.sum(c * c, axis=1, keepdims=True))
        ncn_ref[...] = c / jnp.maximum(cnorm, 1e-12)

    sim = lax.dot_general(
        nx, ncn_ref[...], (((1,), (1,)), ((), ())),
        preferred_element_type=jnp.float32,
    )
    m = sim[:, :_LANES]
    tc = jnp.zeros((_RB, _LANES), jnp.int32)
    for j in range(1, -(-_TILE // _LANES)):
        v = sim[:, j * _LANES:min((j + 1) * _LANES, _TILE)]
        if v.shape[1] < _LANES:
            v = jnp.concatenate(
                [v, jnp.full((_RB, _LANES - v.shape[1]), -jnp.inf,
                             jnp.float32)], axis=1)
        upd = v > m
        m = jnp.where(upd, v, m)
        tc = jnp.where(upd, j, tc)

    tmax = jnp.max(m, axis=1, keepdims=True)
    col = tc * _LANES + lax.broadcasted_iota(jnp.int32, (_RB, _LANES), 1)
    cand = jnp.where(m == tmax, col, jnp.int32(2**31 - 1))
    targ = jnp.min(cand, axis=1, keepdims=True)

    rows = pl.ds(r * _RB, _RB)

    @pl.when(i == 0)
    def _init():
        bv_ref[rows, :] = jnp.full((_RB, 1), -jnp.inf, jnp.float32)
        bi_ref[rows, :] = jnp.zeros((_RB, 1), jnp.int32)

    upd2 = tmax > bv_ref[rows, :]
    bi_ref[rows, :] = jnp.where(upd2, i * _TILE + targ, bi_ref[rows, :])
    bv_ref[rows, :] = jnp.where(upd2, tmax, bv_ref[rows, :])

    @pl.when(i == _NTILES - 1)
    def _fin():
        qidx_ref[rows, :] = bi_ref[rows, :]
        bestsim_ref[rows, :] = bv_ref[rows, :]

    @pl.when((i == _NTILES - 1) & (r == _NRB - 1))
    def _mean():
        simmean_ref[...] = jnp.full(
            (8, _LANES), jnp.sum(bv_ref[...]) / _BATCH, jnp.float32)


def _tc_argmax(x, qx2):
    return pl.pallas_call(
        _argmax_body,
        grid=(_NTILES, _NRB),
        in_specs=[
            pl.BlockSpec((_RB, _DIM), lambda i, r: (r, 0)),
            pl.BlockSpec((_TILE, 2 * _DIM), lambda i, r: (i, 0)),
        ],
        out_specs=[
            pl.BlockSpec((_BATCH, 1), lambda i, r: (0, 0)),
            pl.BlockSpec((_BATCH, 1), lambda i, r: (0, 0)),
            pl.BlockSpec((8, _LANES), lambda i, r: (0, 0)),
        ],
        out_shape=[
            jax.ShapeDtypeStruct((_BATCH, 1), jnp.int32),
            jax.ShapeDtypeStruct((_BATCH, 1), jnp.float32),
            jax.ShapeDtypeStruct((8, _LANES), jnp.float32),
        ],
        scratch_shapes=[
            pltpu.VMEM((_BATCH, 1), jnp.float32),
            pltpu.VMEM((_BATCH, 1), jnp.int32),
            pltpu.VMEM((_TILE, _DIM), jnp.float32),
        ],
    )(x, qx2)


def _sc_gather_body(qx_hbm, age_hbm, idx_hbm, rows_out, age_out, psum_out,
                    idx_v, rows_v, age_v, psum_v, sem):
    wid = lax.axis_index("s") * _NC + lax.axis_index("c")
    base = wid * _BPW
    pltpu.sync_copy(idx_hbm.at[pl.ds(base, _BPW)], idx_v)
    pltpu.async_copy(qx_hbm.at[idx_v], rows_v, sem).wait()
    pltpu.async_copy(age_hbm.at[idx_v], age_v, sem).wait()
    pltpu.sync_copy(rows_v, rows_out.at[pl.ds(base, _BPW)])
    pltpu.sync_copy(age_v, age_out.at[pl.ds(base, _BPW)])
    a = jnp.zeros((16,), jnp.float32)
    for k in range(_BPW // 16):
        a = a + age_v[pl.ds(k * 16, 16)].astype(jnp.float32)
    psum_v[0, :] = a
    pltpu.sync_copy(psum_v, psum_out.at[pl.ds(wid, 1)])


def _sc_gather(queue_x, queue_age, nn_qidx):
    mesh = plsc.VectorSubcoreMesh(core_axis_name="c", subcore_axis_name="s")
    fn = functools.partial(
        pl.kernel,
        mesh=mesh,
        out_type=[
            jax.ShapeDtypeStruct((_BATCH, 2 * _DIM), jnp.float32),
            jax.ShapeDtypeStruct((_BATCH,), jnp.int32),
            jax.ShapeDtypeStruct((_NW, 16), jnp.float32),
        ],
        scratch_types=[
            pltpu.VMEM((_BPW,), jnp.int32),
            pltpu.VMEM((_BPW, 2 * _DIM), jnp.float32),
            pltpu.VMEM((_BPW,), jnp.int32),
            pltpu.VMEM((1, 16), jnp.float32),
            pltpu.SemaphoreType.DMA,
        ],
        compiler_params=pltpu.CompilerParams(use_tc_tiling_on_sc=False),
    )(_sc_gather_body)
    return fn(queue_x, queue_age, nn_qidx)


def kernel(x, idx, queue_x, queue_age):
    qx2 = queue_x.reshape(_SIZE, 2 * _DIM)
    nn_qidx2, best_sim, sim_mean = _tc_argmax(x, qx2)
    nn_qidx = nn_qidx2.reshape(_BATCH)
    rows, nn_age, age_psums = _sc_gather(qx2, queue_age, nn_qidx)
    nn_x = rows[:, :_DIM]
    age_mean = jnp.sum(age_psums) / _BATCH
    return (nn_x, sim_mean[0, 0], age_mean)


# speedup vs baseline: 8.2645x; 1.0024x over previous
import functools

import jax
import jax.numpy as jnp
from jax import lax
from jax.experimental import pallas as pl
from jax.experimental.pallas import tpu as pltpu
from jax.experimental.pallas import tpu_sc as plsc

_SIZE = 100000
_DIM = 32
_BATCH = 1024
_TILE = 10000
_NTILES = _SIZE // _TILE
_LANES = 128
_RB = 1024
_NRB = _BATCH // _RB

_NC = 2
_NS = 16
_NW = _NC * _NS
_BPW = _BATCH // _NW


def _argmax_body(x_ref, cand_ref, qidx_ref, bestsim_ref, simmean_ref,
                 bv_ref, bi_ref, ncn_ref):
    i = pl.program_id(0)
    r = pl.program_id(1)

    xx = x_ref[...]
    xnorm = jnp.sqrt(jnp.sum(xx * xx, axis=1, keepdims=True))
    nx = xx / jnp.maximum(xnorm, 1e-12)

    @pl.when(r == 0)
    def _prep():
        c = cand_ref[:, :_DIM]
        cnorm = jnp.sqrt(jnp.sum(c * c, axis=1, keepdims=True))
        ncn_ref[...] = c / jnp.maximum(cnorm, 1e-12)

    sim = lax.dot_general(
        nx, ncn_ref[...], (((1,), (1,)), ((), ())),
        preferred_element_type=jnp.float32,
    )
    m = sim[:, :_LANES]
    tc = jnp.zeros((_RB, _LANES), jnp.int32)
    for j in range(1, -(-_TILE // _LANES)):
        v = sim[:, j * _LANES:min((j + 1) * _LANES, _TILE)]
        if v.shape[1] < _LANES:
            v = jnp.concatenate(
                [v, jnp.full((_RB, _LANES - v.shape[1]), -jnp.inf,
                             jnp.float32)], axis=1)
        upd = v > m
        m = jnp.where(upd, v, m)
        tc = jnp.where(upd, j, tc)

    tmax = jnp.max(m, axis=1, keepdims=True)
    col = tc * _LANES + lax.broadcasted_iota(jnp.int32, (_RB, _LANES), 1)
    cand = jnp.where(m == tmax, col, jnp.int32(2**31 - 1))
    targ = jnp.min(cand, axis=1, keepdims=True)

    rows = pl.ds(r * _RB, _RB)

    @pl.when(i == 0)
    def _init():
        bv_ref[rows, :] = jnp.full((_RB, 1), -jnp.inf, jnp.float32)
        bi_ref[rows, :] = jnp.zeros((_RB, 1), jnp.int32)

    upd2 = tmax > bv_ref[rows, :]
    bi_ref[rows, :] = jnp.where(upd2, i * _TILE + targ, bi_ref[rows, :])
    bv_ref[rows, :] = jnp.where(upd2, tmax, bv_ref[rows, :])

    @pl.when(i == _NTILES - 1)
    def _fin():
        qidx_ref[rows, :] = bi_ref[rows, :]
        bestsim_ref[rows, :] = bv_ref[rows, :]

    @pl.when((i == _NTILES - 1) & (r == _NRB - 1))
    def _mean():
        simmean_ref[...] = jnp.full(
            (8, _LANES), jnp.sum(bv_ref[...]) / _BATCH, jnp.float32)


def _tc_argmax(x, qx2):
    return pl.pallas_call(
        _argmax_body,
        grid=(_NTILES, _NRB),
        in_specs=[
            pl.BlockSpec((_RB, _DIM), lambda i, r: (r, 0)),
            pl.BlockSpec((_TILE, 2 * _DIM), lambda i, r: (i, 0)),
        ],
        out_specs=[
            pl.BlockSpec((_BATCH, 1), lambda i, r: (0, 0)),
            pl.BlockSpec((_BATCH, 1), lambda i, r: (0, 0)),
            pl.BlockSpec((8, _LANES), lambda i, r: (0, 0)),
        ],
        out_shape=[
            jax.ShapeDtypeStruct((_BATCH, 1), jnp.int32),
            jax.ShapeDtypeStruct((_BATCH, 1), jnp.float32),
            jax.ShapeDtypeStruct((8, _LANES), jnp.float32),
        ],
        scratch_shapes=[
            pltpu.VMEM((_BATCH, 1), jnp.float32),
            pltpu.VMEM((_BATCH, 1), jnp.int32),
            pltpu.VMEM((_TILE, _DIM), jnp.float32),
        ],
    )(x, qx2)


def _sc_gather_body(qx_hbm, age_hbm, idx_hbm, rows_out, psum_out,
                    idx_v, rows_v, age_v, psum_v, sem):
    wid = lax.axis_index("s") * _NC + lax.axis_index("c")
    base = wid * _BPW
    pltpu.sync_copy(idx_hbm.at[pl.ds(base, _BPW)], idx_v)
    pltpu.async_copy(qx_hbm.at[idx_v], rows_v, sem).wait()
    pltpu.async_copy(age_hbm.at[idx_v], age_v, sem).wait()
    pltpu.sync_copy(rows_v.at[:, pl.ds(0, _DIM)],
                    rows_out.at[pl.ds(base, _BPW)])
    a = jnp.zeros((16,), jnp.float32)
    for k in range(_BPW // 16):
        a = a + age_v[pl.ds(k * 16, 16)].astype(jnp.float32)
    psum_v[0, :] = a
    pltpu.sync_copy(psum_v, psum_out.at[pl.ds(wid, 1)])


def _sc_gather(queue_x, queue_age, nn_qidx):
    mesh = plsc.VectorSubcoreMesh(core_axis_name="c", subcore_axis_name="s")
    fn = functools.partial(
        pl.kernel,
        mesh=mesh,
        out_type=[
            jax.ShapeDtypeStruct((_BATCH, _DIM), jnp.float32),
            jax.ShapeDtypeStruct((_NW, 16), jnp.float32),
        ],
        scratch_types=[
            pltpu.VMEM((_BPW,), jnp.int32),
            pltpu.VMEM((_BPW, 2 * _DIM), jnp.float32),
            pltpu.VMEM((_BPW,), jnp.int32),
            pltpu.VMEM((1, 16), jnp.float32),
            pltpu.SemaphoreType.DMA,
        ],
        compiler_params=pltpu.CompilerParams(use_tc_tiling_on_sc=False),
    )(_sc_gather_body)
    return fn(queue_x, queue_age, nn_qidx)


def kernel(x, idx, queue_x, queue_age):
    qx2 = queue_x.reshape(_SIZE, 2 * _DIM)
    nn_qidx2, best_sim, sim_mean = _tc_argmax(x, qx2)
    nn_qidx = nn_qidx2.reshape(_BATCH)
    nn_x, age_psums = _sc_gather(qx2, queue_age, nn_qidx)
    age_mean = jnp.sum(age_psums) / _BATCH
    return (nn_x, sim_mean[0, 0], age_mean)
